# baseline (speedup 1.0000x reference)
import functools
import math

import jax
import jax.numpy as jnp
from jax import lax
from jax.experimental import pallas as pl
from jax.experimental.pallas import tpu as pltpu
from jax.experimental.pallas import tpu_sc as plsc

HEADS = 16
NUM_BUCKETS = 32
MAX_DISTANCE = 128
QLEN = 2048
KLEN = 2048
TW = 4096
SUB = 8
GROUP = 128
LANES = 16
HALF_T = TW // 2

_HALF = NUM_BUCKETS // 2
_MAX_EXACT = _HALF // 2
_THRESH = [
    math.ceil(_MAX_EXACT * (MAX_DISTANCE / _MAX_EXACT) ** (k / (_HALF - _MAX_EXACT)))
    for k in range(1, _HALF - _MAX_EXACT)
]


def _bucket_values(na):
    val_large = jnp.full_like(na, _MAX_EXACT)
    for thresh in _THRESH:
        val_large = val_large + jnp.where(
            na >= jnp.full_like(na, thresh),
            jnp.full_like(na, 1),
            jnp.full_like(na, 0),
        )
    return val_large


def _sc_table_kernel(emb_hbm, off_hbm, out_hbm, emb_v, off_v, row_v):
    wid = lax.axis_index("s") * 2 + lax.axis_index("c")
    h = wid // 2
    half = wid % 2
    pltpu.sync_copy(emb_hbm, emb_v)
    pltpu.sync_copy(off_hbm, off_v)
    def body(it, _):
        offset = off_v[...]
        iota = jax.lax.broadcasted_iota(jnp.int32, (LANES,), 0)
        h_vec = jnp.full((LANES,), h, jnp.int32)
        base = half * HALF_T - (QLEN - 1) + it * LANES
        n = -(iota + jnp.full((LANES,), base, jnp.int32) - offset)
        zero = jnp.zeros((LANES,), jnp.int32)
        ret = jnp.where(n < zero, jnp.full_like(n, _HALF), zero)
        na = jnp.abs(n)
        bucket = ret + jnp.where(
            na < jnp.full_like(na, _MAX_EXACT), na, _bucket_values(na)
        )
        idx = bucket * jnp.full_like(bucket, HEADS) + h_vec
        row_v[pl.ds(it * LANES, LANES)] = plsc.load_gather(emb_v, [idx])
        return 0

    lax.fori_loop(0, HALF_T // LANES, body, 0)
    pltpu.sync_copy(row_v, out_hbm.at[h * 2 + half])


_sc_table = functools.partial(
    pl.kernel,
    mesh=plsc.VectorSubcoreMesh(core_axis_name="c", subcore_axis_name="s"),
    compiler_params=pltpu.CompilerParams(needs_layout_passes=False),
    out_type=jax.ShapeDtypeStruct((2 * HEADS, HALF_T), jnp.float32),
    scratch_types=[
        pltpu.VMEM((NUM_BUCKETS * HEADS,), jnp.float32),
        pltpu.VMEM((LANES,), jnp.int32),
        pltpu.VMEM((HALF_T,), jnp.float32),
    ],
)(_sc_table_kernel)


def _tc_expand_kernel(vtab_ref, out_ref, vs_ref):
    V = vtab_ref[0]
    si = jax.lax.broadcasted_iota(jnp.int32, (SUB, TW), 0)
    shifts = (SUB - 1) - si
    Vs8 = jnp.broadcast_to(V, (SUB, TW))
    bit = 1
    while bit < SUB:
        rolled = pltpu.roll(Vs8, TW - bit, 1)
        Vs8 = jnp.where((shifts & bit) != 0, rolled, Vs8)
        bit *= 2

    for a in range(GROUP // SUB):
        shift = SUB * (GROUP // SUB - 1 - a)
        vs_ref[SUB * a : SUB * (a + 1), :] = pltpu.roll(Vs8, (TW - shift) % TW, 1)

    for g in range(QLEN // GROUP):
        s = (QLEN - 1) - (GROUP - 1) - GROUP * g
        out_ref[0, GROUP * g : GROUP * (g + 1), :] = vs_ref[:, s : s + KLEN]


def kernel(qlen, klen, emb):
    offset = (jnp.asarray(klen) - jnp.asarray(qlen)).astype(jnp.int32)
    off_arr = jnp.full((LANES,), 0, jnp.int32) + offset

    vtab = _sc_table(emb.reshape(-1), off_arr)
    vtab3 = vtab.reshape(HEADS, 1, TW)

    out = pl.pallas_call(
        _tc_expand_kernel,
        grid=(HEADS,),
        in_specs=[pl.BlockSpec((1, 1, TW), lambda h: (h, 0, 0))],
        out_specs=pl.BlockSpec((1, QLEN, KLEN), lambda h: (h, 0, 0)),
        out_shape=jax.ShapeDtypeStruct((HEADS, QLEN, KLEN), jnp.float32),
        scratch_shapes=[pltpu.VMEM((GROUP, TW), jnp.float32)],
    )(vtab3)
    return out


# speedup vs baseline: 1.2943x; 1.2943x over previous
import math

import jax
import jax.numpy as jnp
from jax.experimental import pallas as pl
from jax.experimental.pallas import tpu as pltpu

HEADS = 16
NUM_BUCKETS = 32
MAX_DISTANCE = 128
QLEN = 2048
KLEN = 2048
TW = 4096
SUB = 8
GROUP = 128


def _bias_kernel(off_ref, embT_ref, out_ref, vs_ref):
    offset = off_ref[0]

    lane = jax.lax.broadcasted_iota(jnp.int32, (SUB, TW), 1)
    sub = jax.lax.broadcasted_iota(jnp.int32, (SUB, TW), 0)
    t = lane + (SUB - 1) - sub
    d = t - (QLEN - 1) - offset
    n = -d
    half = NUM_BUCKETS // 2
    ret = jnp.where(n < 0, half, 0)
    na = jnp.abs(n)
    max_exact = half // 2
    val_large = jnp.full_like(na, max_exact)
    for k in range(1, half - max_exact):
        thresh = max_exact * (MAX_DISTANCE / max_exact) ** (k / (half - max_exact))
        val_large = val_large + (na >= math.ceil(thresh)).astype(jnp.int32)
    bucket = ret + jnp.where(na < max_exact, na, val_large)

    Vs8 = jnp.zeros((SUB, TW), jnp.float32)
    for b in range(NUM_BUCKETS):
        Vs8 = Vs8 + jnp.where(bucket == b, embT_ref[0, 0:1, b : b + 1], 0.0)

    for a in range(GROUP // SUB):
        shift = SUB * (GROUP // SUB - 1 - a)
        vs_ref[SUB * a : SUB * (a + 1), :] = pltpu.roll(Vs8, (TW - shift) % TW, 1)

    for g in range(QLEN // GROUP):
        s = (QLEN - 1) - (GROUP - 1) - GROUP * g
        out_ref[0, GROUP * g : GROUP * (g + 1), :] = vs_ref[:, s : s + KLEN]


def kernel(qlen, klen, emb):
    offset = (jnp.asarray(klen) - jnp.asarray(qlen)).astype(jnp.int32)
    off = jnp.reshape(offset, (1,))
    embT = emb.T.reshape(HEADS, 1, NUM_BUCKETS)

    out = pl.pallas_call(
        _bias_kernel,
        grid=(HEADS,),
        in_specs=[
            pl.BlockSpec(memory_space=pltpu.SMEM),
            pl.BlockSpec((1, 1, NUM_BUCKETS), lambda h: (h, 0, 0)),
        ],
        out_specs=pl.BlockSpec((1, QLEN, KLEN), lambda h: (h, 0, 0)),
        out_shape=jax.ShapeDtypeStruct((HEADS, QLEN, KLEN), jnp.float32),
        scratch_shapes=[pltpu.VMEM((GROUP, TW), jnp.float32)],
    )(off, embT)
    return out
